# Initial kernel scaffold; baseline (speedup 1.0000x reference)
import jax
import jax.numpy as jnp
from jax.experimental import pallas as pl


def kernel(x, W_f, b_f, Wc, bc):
    raise NotImplementedError("write your pallas kernel here")


# speedup vs baseline: 1.2498x; 1.2498x over previous
import functools
import jax
import jax.numpy as jnp
from jax.experimental import pallas as pl
from jax.experimental.pallas import tpu as pltpu

B = 4096
D = 512
C = 1000
CP = 1024
N = C + B
NP = 5120
FILTER_K = 100
LAM = 0.1
KN = 3
NEG = -1e30
EPS = 1e-12


def _a_body(x_ref, w_ref, b_ref, z_ref):
    z_ref[...] = jnp.dot(x_ref[...], w_ref[...],
                         preferred_element_type=jnp.float32) + b_ref[...]


def _b_body(s_ref, wc_ref, bc_ref,
            pd_ref, sc_ref, sn2_ref, sn1_ref, ent_ref, cls_ref, s2_ref):
    s = s_ref[...]
    p = jax.lax.dot_general(s, wc_ref[...], (((1,), (1,)), ((), ())),
                            preferred_element_type=jnp.float32) + bc_ref[...]
    col = jax.lax.broadcasted_iota(jnp.int32, p.shape, 1)
    m = jnp.max(p, axis=1, keepdims=True)
    e = jnp.exp(p - m)
    se = jnp.sum(e, axis=1, keepdims=True)
    scores = e / se
    ls = p - (m + jnp.log(se))
    ent = -jnp.sum(scores * ls, axis=1, keepdims=True)
    cls = jnp.min(jnp.where(p == m, col, CP), axis=1, keepdims=True)
    pd = jnp.where(col < C, p, 0.0)
    n2 = jnp.sqrt(jnp.sum(s * s, axis=1, keepdims=True))
    sn2 = s / jnp.maximum(n2, EPS)
    n1 = jnp.sum(jnp.abs(sn2), axis=1, keepdims=True)
    sn1 = sn2 / jnp.maximum(n1, EPS)
    pd_ref[...] = pd
    sc_ref[...] = scores
    sn2_ref[...] = sn2
    sn1_ref[...] = sn1
    ent_ref[...] = ent.reshape(1, 1, 128)
    cls_ref[...] = cls.astype(jnp.int32).reshape(1, 1, 128)
    s2_ref[...] = jnp.sum(scores * scores, axis=1, keepdims=True).reshape(1, 1, 128)


def _c_body(enti_ref, clsi_ref, entr_ref, clsr_ref, keep_ref, si_ref):
    i = pl.program_id(0)
    ei = enti_ref[0, 0, :].reshape(128, 1)
    ci = clsi_ref[0, 0, :].reshape(128, 1)
    gi = i * 128 + jax.lax.broadcasted_iota(jnp.int32, (128, 1), 0)
    er = entr_ref[...].reshape(1, NP)
    cr = clsr_ref[...].reshape(1, NP)
    gj = jax.lax.broadcasted_iota(jnp.int32, (1, NP), 1)
    lt = (er < ei) | ((er == ei) & (gj < gi))
    cnt = jnp.sum(jnp.where((cr == ci) & lt & (gj < N), 1, 0), axis=1,
                  keepdims=True)
    keep = (cnt < FILTER_K) & (gi < N)
    keep_ref[...] = keep.astype(jnp.int32).reshape(1, 1, 128)
    si_ref[...] = jnp.where(keep, ci, C).astype(jnp.int32).reshape(1, 1, 128)


def _d_body(sn2_ref, si_ref, pn_ref, acc_ref):
    j = pl.program_id(0)
    si = si_ref[0, 0, :].reshape(128, 1)
    oh = (si == jax.lax.broadcasted_iota(jnp.int32, (128, CP), 1)
          ).astype(jnp.float32)
    contrib = jax.lax.dot_general(oh, sn2_ref[...], (((0,), (0,)), ((), ())),
                                  preferred_element_type=jnp.float32)

    @pl.when(j == 0)
    def _():
        acc_ref[...] = jnp.zeros_like(acc_ref)

    acc_ref[...] += contrib

    @pl.when(j == pl.num_programs(0) - 1)
    def _():
        w = acc_ref[...]
        n1 = jnp.sum(jnp.abs(w), axis=1, keepdims=True)
        pn_ref[...] = w / jnp.maximum(n1, EPS)


def _top3_merge(cand, idmap):
    col = jax.lax.broadcasted_iota(jnp.int32, cand.shape, 1)
    vals, ids = [], []
    for _ in range(KN):
        m = jnp.max(cand, axis=1, keepdims=True)
        jsel = jnp.min(jnp.where(cand == m, col, 2 ** 30), axis=1,
                       keepdims=True)
        hit = col == jsel
        ids.append(jnp.sum(jnp.where(hit, idmap, 0), axis=1, keepdims=True))
        vals.append(m)
        cand = jnp.where(hit, NEG, cand)
    return jnp.concatenate(vals, axis=1), jnp.concatenate(ids, axis=1)


def _e_body(zn_ref, sn1_ref, keep_ref, val_ref, idx_ref, vs_ref, is_ref):
    j = pl.program_id(1)

    @pl.when(j == 0)
    def _():
        vs_ref[...] = jnp.full_like(vs_ref, NEG)
        is_ref[...] = jnp.zeros_like(is_ref)

    sim = jax.lax.dot_general(zn_ref[...], sn1_ref[...],
                              (((1,), (1,)), ((), ())),
                              preferred_element_type=jnp.float32)
    keep = keep_ref[...].reshape(1, 512)
    sim = jnp.where(keep == 1, sim, NEG)
    jid = j * 512 + jax.lax.broadcasted_iota(jnp.int32, sim.shape, 1)
    cand = jnp.concatenate([vs_ref[:, :KN], sim], axis=1)
    idmap = jnp.concatenate([is_ref[:, :KN], jid], axis=1)
    v3, i3 = _top3_merge(cand, idmap)
    vs_ref[:, :KN] = v3
    is_ref[:, :KN] = i3

    @pl.when(j == pl.num_programs(1) - 1)
    def _():
        val_ref[...] = vs_ref[...]
        idx_ref[...] = is_ref[...]


def _f_body(pz_ref, scr_ref, s2_ref, idx_ref, val_ref, out_ref, acc_ref):
    j = pl.program_id(1)

    @pl.when(j == 0)
    def _():
        acc_ref[...] = jnp.zeros_like(acc_ref)

    g = jax.lax.dot_general(pz_ref[...], scr_ref[...],
                            (((1,), (1,)), ((), ())),
                            preferred_element_type=jnp.float32)
    jid = j * 512 + jax.lax.broadcasted_iota(jnp.int32, g.shape, 1)
    s2 = s2_ref[...].reshape(1, 512)
    for kk in range(KN):
        tid = idx_ref[:, kk:kk + 1]
        hit = jid == tid
        acc_ref[:, kk:kk + 1] += jnp.sum(jnp.where(hit, g, 0.0), axis=1,
                                         keepdims=True)
        acc_ref[:, KN + kk:KN + kk + 1] += jnp.sum(
            jnp.where(hit, s2, 0.0), axis=1, keepdims=True)

    @pl.when(j == pl.num_programs(1) - 1)
    def _():
        pz = pz_ref[...]
        p2 = jnp.sum(pz * pz, axis=1, keepdims=True)
        tot = jnp.zeros((256, 1), jnp.float32)
        for kk in range(KN):
            diff = p2 + acc_ref[:, KN + kk:KN + kk + 1] \
                - 2.0 * acc_ref[:, kk:kk + 1]
            tot = tot + (-val_ref[:, kk:kk + 1]) * diff
        out_ref[...] = tot.reshape(1, 1, 256)


def _g_body(zn_ref, pn_ref, scr_ref, out_ref):
    dist = jax.lax.dot_general(zn_ref[...], pn_ref[...],
                               (((1,), (1,)), ((), ())),
                               preferred_element_type=jnp.float32)
    col = jax.lax.broadcasted_iota(jnp.int32, dist.shape, 1)
    dist = jnp.where(col < C, dist, NEG)
    m = jnp.max(dist, axis=1, keepdims=True)
    e = jnp.exp(dist - m)
    tgt = e / jnp.sum(e, axis=1, keepdims=True)
    s = jnp.where(col < C, scr_ref[...], NEG)
    m2 = jnp.max(s, axis=1, keepdims=True)
    e2 = jnp.where(col < C, jnp.exp(s - m2), 0.0)
    ls2 = s - (m2 + jnp.log(jnp.sum(e2, axis=1, keepdims=True)))
    kl = tgt * (jnp.log(jnp.clip(tgt, 1e-38, None)) - ls2)
    out_ref[...] = jnp.sum(jnp.where(col < C, kl, 0.0), axis=1).reshape(1, 1, 256)


def kernel(x, W_f, b_f, Wc, bc):
    f32 = jnp.float32
    z = pl.pallas_call(
        _a_body,
        grid=(16,),
        in_specs=[pl.BlockSpec((256, D), lambda i: (i, 0)),
                  pl.BlockSpec((D, D), lambda i: (0, 0)),
                  pl.BlockSpec((1, D), lambda i: (0, 0))],
        out_specs=pl.BlockSpec((256, D), lambda i: (i, 0)),
        out_shape=jax.ShapeDtypeStruct((B, D), f32),
    )(x, W_f, b_f.reshape(1, D))

    S = jnp.concatenate([Wc, z, jnp.zeros((NP - N, D), f32)], axis=0)
    Wcp = jnp.concatenate([Wc, jnp.zeros((CP - C, D), f32)], axis=0)
    bcp = jnp.concatenate([bc, jnp.full((CP - C,), NEG, f32)],
                          axis=0).reshape(1, CP)

    nb = NP // 128
    pd, scores, sn2, sn1, ent3, cls3, s23 = pl.pallas_call(
        _b_body,
        grid=(nb,),
        in_specs=[pl.BlockSpec((128, D), lambda i: (i, 0)),
                  pl.BlockSpec((CP, D), lambda i: (0, 0)),
                  pl.BlockSpec((1, CP), lambda i: (0, 0))],
        out_specs=[pl.BlockSpec((128, CP), lambda i: (i, 0)),
                   pl.BlockSpec((128, CP), lambda i: (i, 0)),
                   pl.BlockSpec((128, D), lambda i: (i, 0)),
                   pl.BlockSpec((128, D), lambda i: (i, 0)),
                   pl.BlockSpec((1, 1, 128), lambda i: (i, 0, 0)),
                   pl.BlockSpec((1, 1, 128), lambda i: (i, 0, 0)),
                   pl.BlockSpec((1, 1, 128), lambda i: (i, 0, 0))],
        out_shape=[jax.ShapeDtypeStruct((NP, CP), f32),
                   jax.ShapeDtypeStruct((NP, CP), f32),
                   jax.ShapeDtypeStruct((NP, D), f32),
                   jax.ShapeDtypeStruct((NP, D), f32),
                   jax.ShapeDtypeStruct((nb, 1, 128), f32),
                   jax.ShapeDtypeStruct((nb, 1, 128), jnp.int32),
                   jax.ShapeDtypeStruct((nb, 1, 128), f32)],
    )(S, Wcp, bcp)

    ent_row = ent3.reshape(1, NP)
    cls_row = cls3.reshape(1, NP)
    keep3, si3 = pl.pallas_call(
        _c_body,
        grid=(nb,),
        in_specs=[pl.BlockSpec((1, 1, 128), lambda i: (i, 0, 0)),
                  pl.BlockSpec((1, 1, 128), lambda i: (i, 0, 0)),
                  pl.BlockSpec((1, NP), lambda i: (0, 0)),
                  pl.BlockSpec((1, NP), lambda i: (0, 0))],
        out_specs=[pl.BlockSpec((1, 1, 128), lambda i: (i, 0, 0)),
                   pl.BlockSpec((1, 1, 128), lambda i: (i, 0, 0))],
        out_shape=[jax.ShapeDtypeStruct((nb, 1, 128), jnp.int32),
                   jax.ShapeDtypeStruct((nb, 1, 128), jnp.int32)],
    )(ent3, cls3, ent_row, cls_row)

    pn = pl.pallas_call(
        _d_body,
        grid=(nb,),
        in_specs=[pl.BlockSpec((128, D), lambda j: (j, 0)),
                  pl.BlockSpec((1, 1, 128), lambda j: (j, 0, 0))],
        out_specs=pl.BlockSpec((CP, D), lambda j: (0, 0)),
        out_shape=jax.ShapeDtypeStruct((CP, D), f32),
        scratch_shapes=[pltpu.VMEM((CP, D), f32)],
        compiler_params=pltpu.CompilerParams(
            dimension_semantics=("arbitrary",)),
    )(sn2, si3)

    zn = jax.lax.slice(sn1, (C, 0), (N, D))
    keepE = keep3.reshape(10, 1, 512)

    val128, idx128 = pl.pallas_call(
        _e_body,
        grid=(16, 10),
        in_specs=[pl.BlockSpec((256, D), lambda i, j: (i, 0)),
                  pl.BlockSpec((512, D), lambda i, j: (j, 0)),
                  pl.BlockSpec((1, 1, 512), lambda i, j: (j, 0, 0))],
        out_specs=[pl.BlockSpec((256, 128), lambda i, j: (i, 0)),
                   pl.BlockSpec((256, 128), lambda i, j: (i, 0))],
        out_shape=[jax.ShapeDtypeStruct((B, 128), f32),
                   jax.ShapeDtypeStruct((B, 128), jnp.int32)],
        scratch_shapes=[pltpu.VMEM((256, 128), f32),
                        pltpu.VMEM((256, 128), jnp.int32)],
        compiler_params=pltpu.CompilerParams(
            dimension_semantics=("parallel", "arbitrary")),
    )(zn, sn1, keepE)

    pz = jax.lax.slice(pd, (C, 0), (N, CP))
    s2row = s23.reshape(10, 1, 512)
    lloc = pl.pallas_call(
        _f_body,
        grid=(16, 10),
        in_specs=[pl.BlockSpec((256, CP), lambda i, j: (i, 0)),
                  pl.BlockSpec((512, CP), lambda i, j: (j, 0)),
                  pl.BlockSpec((1, 1, 512), lambda i, j: (j, 0, 0)),
                  pl.BlockSpec((256, 128), lambda i, j: (i, 0)),
                  pl.BlockSpec((256, 128), lambda i, j: (i, 0))],
        out_specs=pl.BlockSpec((1, 1, 256), lambda i, j: (i, 0, 0)),
        out_shape=jax.ShapeDtypeStruct((16, 1, 256), f32),
        scratch_shapes=[pltpu.VMEM((256, 128), f32)],
        compiler_params=pltpu.CompilerParams(
            dimension_semantics=("parallel", "arbitrary")),
    )(pz, scores, s2row, idx128, val128)

    scz = jax.lax.slice(scores, (C, 0), (N, CP))
    klrows = pl.pallas_call(
        _g_body,
        grid=(16,),
        in_specs=[pl.BlockSpec((256, D), lambda i: (i, 0)),
                  pl.BlockSpec((CP, D), lambda i: (0, 0)),
                  pl.BlockSpec((256, CP), lambda i: (i, 0))],
        out_specs=pl.BlockSpec((1, 1, 256), lambda i: (i, 0, 0)),
        out_shape=jax.ShapeDtypeStruct((16, 1, 256), f32),
    )(zn, pn, scz)

    loss = jnp.sum(klrows) / B + LAM * (jnp.sum(lloc) / (B * KN))
    p_out = jax.lax.slice(pd, (C, 0), (N, C))
    return p_out, loss


# speedup vs baseline: 1.6949x; 1.3561x over previous
import functools
import jax
import jax.numpy as jnp
from jax import lax
from jax.experimental import pallas as pl
from jax.experimental.pallas import tpu as pltpu
from jax.experimental.pallas import tpu_sc as plsc

B = 4096
D = 512
C = 1000
CP = 1024
N = C + B
NP = 5120
FILTER_K = 100
LAM = 0.1
KN = 3
NEG = -1e30
EPS = 1e-12


def _a_body(x_ref, w_ref, b_ref, z_ref):
    z_ref[...] = jnp.dot(x_ref[...], w_ref[...],
                         preferred_element_type=jnp.float32) + b_ref[...]


def _b_body(s_ref, wc_ref, bc_ref,
            pd_ref, sc_ref, sn2_ref, sn1_ref, ent_ref, cls_ref, s2_ref):
    s = s_ref[...]
    p = jax.lax.dot_general(s, wc_ref[...], (((1,), (1,)), ((), ())),
                            preferred_element_type=jnp.float32) + bc_ref[...]
    col = jax.lax.broadcasted_iota(jnp.int32, p.shape, 1)
    m = jnp.max(p, axis=1, keepdims=True)
    e = jnp.exp(p - m)
    se = jnp.sum(e, axis=1, keepdims=True)
    scores = e / se
    ls = p - (m + jnp.log(se))
    ent = -jnp.sum(scores * ls, axis=1, keepdims=True)
    cls = jnp.min(jnp.where(p == m, col, CP), axis=1, keepdims=True)
    pd = jnp.where(col < C, p, 0.0)
    n2 = jnp.sqrt(jnp.sum(s * s, axis=1, keepdims=True))
    sn2 = s / jnp.maximum(n2, EPS)
    n1 = jnp.sum(jnp.abs(sn2), axis=1, keepdims=True)
    sn1 = sn2 / jnp.maximum(n1, EPS)
    pd_ref[...] = pd
    sc_ref[...] = scores
    sn2_ref[...] = sn2
    sn1_ref[...] = sn1
    ent_ref[...] = ent.reshape(1, 1, 128)
    cls_ref[...] = cls.astype(jnp.int32).reshape(1, 1, 128)
    s2_ref[...] = jnp.sum(scores * scores, axis=1, keepdims=True).reshape(1, 1, 128)


def _c_body(enti_ref, clsi_ref, entr_ref, clsr_ref, keep_ref, si_ref):
    i = pl.program_id(0)
    ei = enti_ref[0, 0, :].reshape(128, 1)
    ci = clsi_ref[0, 0, :].reshape(128, 1)
    gi = i * 128 + jax.lax.broadcasted_iota(jnp.int32, (128, 1), 0)
    er = entr_ref[...].reshape(1, NP)
    cr = clsr_ref[...].reshape(1, NP)
    gj = jax.lax.broadcasted_iota(jnp.int32, (1, NP), 1)
    lt = (er < ei) | ((er == ei) & (gj < gi))
    cnt = jnp.sum(jnp.where((cr == ci) & lt & (gj < N), 1, 0), axis=1,
                  keepdims=True)
    keep = (cnt < FILTER_K) & (gi < N)
    keep_ref[...] = keep.astype(jnp.int32).reshape(1, 1, 128)
    si_ref[...] = jnp.where(keep, ci, C).astype(jnp.int32).reshape(1, 1, 128)


NC = 2
NS = 16
NW = NC * NS

ROWS_W = NP // NW
IW = B // NW
ICH = 16
PCH = ICH * KN


def _f_sc(pz_hbm, scores_hbm, idx_hbm, out_hbm, idx_v, prows, srows, obuf, sem):
    cid = lax.axis_index("c")
    sid = lax.axis_index("s")
    wid = sid * NC + cid
    base_i = wid * IW
    base_p = wid * IW * KN
    pltpu.sync_copy(idx_hbm.at[pl.ds(base_p, IW * KN)], idx_v)

    def chunk(cc, _):
        pltpu.sync_copy(pz_hbm.at[pl.ds(base_i + cc * ICH, ICH)], prows)
        pltpu.async_copy(scores_hbm.at[idx_v.at[pl.ds(cc * PCH, PCH)]],
                         srows, sem).wait()

        def irow(ii, _):
            def cstep(c, d):
                pv = prows[ii, pl.ds(c * 16, 16)]
                r = []
                for kk in range(KN):
                    sv = srows[ii * KN + kk, pl.ds(c * 16, 16)]
                    t = pv - sv
                    r.append(d[kk] + t * t)
                return tuple(r)

            z16 = jnp.zeros((16,), jnp.float32)
            d = lax.fori_loop(0, CP // 16, cstep, (z16, z16, z16))
            for kk in range(KN):
                obuf[cc * PCH + ii * KN + kk, :] = d[kk]
            return 0

        lax.fori_loop(0, ICH, irow, 0)
        return 0

    lax.fori_loop(0, IW // ICH, chunk, 0)
    pltpu.sync_copy(obuf, out_hbm.at[pl.ds(base_p, IW * KN)])


def _d_body(sn2_ref, si_ref, pn_ref, acc_ref):
    j = pl.program_id(0)
    si = si_ref[0, 0, :].reshape(128, 1)
    oh = (si == jax.lax.broadcasted_iota(jnp.int32, (128, CP), 1)
          ).astype(jnp.float32)
    contrib = jax.lax.dot_general(oh, sn2_ref[...], (((0,), (0,)), ((), ())),
                                  preferred_element_type=jnp.float32)

    @pl.when(j == 0)
    def _():
        acc_ref[...] = jnp.zeros_like(acc_ref)

    acc_ref[...] += contrib

    @pl.when(j == pl.num_programs(0) - 1)
    def _():
        w = acc_ref[...]
        n1 = jnp.sum(jnp.abs(w), axis=1, keepdims=True)
        pn_ref[...] = w / jnp.maximum(n1, EPS)


def _h_body(dacc_ref, val_ref, kl_ref, out_ref):
    tot = jnp.zeros((B, 1), jnp.float32)
    for kk in range(KN):
        diff = jnp.sum(dacc_ref[:, kk * 16:(kk + 1) * 16], axis=1,
                       keepdims=True)
        tot = tot + (-val_ref[:, kk:kk + 1]) * diff
    out_ref[...] = (jnp.sum(kl_ref[...]) / B
                    + LAM * jnp.sum(tot) / (B * KN)).reshape(1, 1)


def _top3_merge(cand, idmap):
    col = jax.lax.broadcasted_iota(jnp.int32, cand.shape, 1)
    vals, ids = [], []
    for _ in range(KN):
        m = jnp.max(cand, axis=1, keepdims=True)
        jsel = jnp.min(jnp.where(cand == m, col, 2 ** 30), axis=1,
                       keepdims=True)
        hit = col == jsel
        ids.append(jnp.sum(jnp.where(hit, idmap, 0), axis=1, keepdims=True))
        vals.append(m)
        cand = jnp.where(hit, NEG, cand)
    return jnp.concatenate(vals, axis=1), jnp.concatenate(ids, axis=1)


def _e_body(zn_ref, sn1_ref, keep_ref, val_ref, idx_ref, vs_ref, is_ref):
    j = pl.program_id(1)

    @pl.when(j == 0)
    def _():
        vs_ref[...] = jnp.full_like(vs_ref, NEG)
        is_ref[...] = jnp.zeros_like(is_ref)

    sim = jax.lax.dot_general(zn_ref[...], sn1_ref[...],
                              (((1,), (1,)), ((), ())),
                              preferred_element_type=jnp.float32)
    keep = keep_ref[...].reshape(1, 512)
    sim = jnp.where(keep == 1, sim, NEG)
    jid = j * 512 + jax.lax.broadcasted_iota(jnp.int32, sim.shape, 1)
    cand = jnp.concatenate([vs_ref[:, :KN], sim], axis=1)
    idmap = jnp.concatenate([is_ref[:, :KN], jid], axis=1)
    v3, i3 = _top3_merge(cand, idmap)
    vs_ref[:, :KN] = v3
    is_ref[:, :KN] = i3

    @pl.when(j == pl.num_programs(1) - 1)
    def _():
        val_ref[...] = vs_ref[...]
        idx_ref[...] = is_ref[...]


def _g_body(zn_ref, pn_ref, scr_ref, out_ref):
    dist = jax.lax.dot_general(zn_ref[...], pn_ref[...],
                               (((1,), (1,)), ((), ())),
                               preferred_element_type=jnp.float32)
    col = jax.lax.broadcasted_iota(jnp.int32, dist.shape, 1)
    dist = jnp.where(col < C, dist, NEG)
    m = jnp.max(dist, axis=1, keepdims=True)
    e = jnp.exp(dist - m)
    tgt = e / jnp.sum(e, axis=1, keepdims=True)
    s = jnp.where(col < C, scr_ref[...], NEG)
    m2 = jnp.max(s, axis=1, keepdims=True)
    e2 = jnp.where(col < C, jnp.exp(s - m2), 0.0)
    ls2 = s - (m2 + jnp.log(jnp.sum(e2, axis=1, keepdims=True)))
    kl = tgt * (jnp.log(jnp.clip(tgt, 1e-38, None)) - ls2)
    out_ref[...] = jnp.sum(jnp.where(col < C, kl, 0.0), axis=1).reshape(1, 1, 256)


def kernel(x, W_f, b_f, Wc, bc):
    f32 = jnp.float32
    z = pl.pallas_call(
        _a_body,
        grid=(16,),
        in_specs=[pl.BlockSpec((256, D), lambda i: (i, 0)),
                  pl.BlockSpec((D, D), lambda i: (0, 0)),
                  pl.BlockSpec((1, D), lambda i: (0, 0))],
        out_specs=pl.BlockSpec((256, D), lambda i: (i, 0)),
        out_shape=jax.ShapeDtypeStruct((B, D), f32),
    )(x, W_f, b_f.reshape(1, D))

    S = jnp.concatenate([Wc, z, jnp.zeros((NP - N, D), f32)], axis=0)
    Wcp = jnp.concatenate([Wc, jnp.zeros((CP - C, D), f32)], axis=0)
    bcp = jnp.concatenate([bc, jnp.full((CP - C,), NEG, f32)],
                          axis=0).reshape(1, CP)

    nb = NP // 128
    pd, scores, sn2, sn1, ent3, cls3, s23 = pl.pallas_call(
        _b_body,
        grid=(nb,),
        in_specs=[pl.BlockSpec((128, D), lambda i: (i, 0)),
                  pl.BlockSpec((CP, D), lambda i: (0, 0)),
                  pl.BlockSpec((1, CP), lambda i: (0, 0))],
        out_specs=[pl.BlockSpec((128, CP), lambda i: (i, 0)),
                   pl.BlockSpec((128, CP), lambda i: (i, 0)),
                   pl.BlockSpec((128, D), lambda i: (i, 0)),
                   pl.BlockSpec((128, D), lambda i: (i, 0)),
                   pl.BlockSpec((1, 1, 128), lambda i: (i, 0, 0)),
                   pl.BlockSpec((1, 1, 128), lambda i: (i, 0, 0)),
                   pl.BlockSpec((1, 1, 128), lambda i: (i, 0, 0))],
        out_shape=[jax.ShapeDtypeStruct((NP, CP), f32),
                   jax.ShapeDtypeStruct((NP, CP), f32),
                   jax.ShapeDtypeStruct((NP, D), f32),
                   jax.ShapeDtypeStruct((NP, D), f32),
                   jax.ShapeDtypeStruct((nb, 1, 128), f32),
                   jax.ShapeDtypeStruct((nb, 1, 128), jnp.int32),
                   jax.ShapeDtypeStruct((nb, 1, 128), f32)],
    )(S, Wcp, bcp)

    ent_row = ent3.reshape(1, NP)
    cls_row = cls3.reshape(1, NP)
    keep3, si3 = pl.pallas_call(
        _c_body,
        grid=(nb,),
        in_specs=[pl.BlockSpec((1, 1, 128), lambda i: (i, 0, 0)),
                  pl.BlockSpec((1, 1, 128), lambda i: (i, 0, 0)),
                  pl.BlockSpec((1, NP), lambda i: (0, 0)),
                  pl.BlockSpec((1, NP), lambda i: (0, 0))],
        out_specs=[pl.BlockSpec((1, 1, 128), lambda i: (i, 0, 0)),
                   pl.BlockSpec((1, 1, 128), lambda i: (i, 0, 0))],
        out_shape=[jax.ShapeDtypeStruct((nb, 1, 128), jnp.int32),
                   jax.ShapeDtypeStruct((nb, 1, 128), jnp.int32)],
    )(ent3, cls3, ent_row, cls_row)

    mesh = plsc.VectorSubcoreMesh(core_axis_name="c", subcore_axis_name="s")
    pn = pl.pallas_call(
        _d_body,
        grid=(nb,),
        in_specs=[pl.BlockSpec((128, D), lambda j: (j, 0)),
                  pl.BlockSpec((1, 1, 128), lambda j: (j, 0, 0))],
        out_specs=pl.BlockSpec((CP, D), lambda j: (0, 0)),
        out_shape=jax.ShapeDtypeStruct((CP, D), f32),
        scratch_shapes=[pltpu.VMEM((CP, D), f32)],
        compiler_params=pltpu.CompilerParams(
            dimension_semantics=("arbitrary",)),
    )(sn2, si3)

    zn = jax.lax.slice(sn1, (C, 0), (N, D))
    keepE = keep3.reshape(10, 1, 512)

    val128, idx128 = pl.pallas_call(
        _e_body,
        grid=(16, 10),
        in_specs=[pl.BlockSpec((256, D), lambda i, j: (i, 0)),
                  pl.BlockSpec((512, D), lambda i, j: (j, 0)),
                  pl.BlockSpec((1, 1, 512), lambda i, j: (j, 0, 0))],
        out_specs=[pl.BlockSpec((256, 128), lambda i, j: (i, 0)),
                   pl.BlockSpec((256, 128), lambda i, j: (i, 0))],
        out_shape=[jax.ShapeDtypeStruct((B, 128), f32),
                   jax.ShapeDtypeStruct((B, 128), jnp.int32)],
        scratch_shapes=[pltpu.VMEM((256, 128), f32),
                        pltpu.VMEM((256, 128), jnp.int32)],
        compiler_params=pltpu.CompilerParams(
            dimension_semantics=("parallel", "arbitrary")),
    )(zn, sn1, keepE)

    pz = jax.lax.slice(pd, (C, 0), (N, CP))
    idx_flat = idx128[:, :KN].reshape(B * KN)
    f_call = functools.partial(
        pl.kernel,
        mesh=mesh,
        out_type=jax.ShapeDtypeStruct((B * KN, 16), f32),
        scratch_types=[pltpu.VMEM((IW * KN,), jnp.int32),
                       pltpu.VMEM((ICH, CP), f32),
                       pltpu.VMEM((PCH, CP), f32),
                       pltpu.VMEM((IW * KN, 16), f32),
                       pltpu.SemaphoreType.DMA],
    )(_f_sc)
    dacc = f_call(pz, scores, idx_flat)

    scz = jax.lax.slice(scores, (C, 0), (N, CP))
    klrows = pl.pallas_call(
        _g_body,
        grid=(16,),
        in_specs=[pl.BlockSpec((256, D), lambda i: (i, 0)),
                  pl.BlockSpec((CP, D), lambda i: (0, 0)),
                  pl.BlockSpec((256, CP), lambda i: (i, 0))],
        out_specs=pl.BlockSpec((1, 1, 256), lambda i: (i, 0, 0)),
        out_shape=jax.ShapeDtypeStruct((16, 1, 256), f32),
    )(zn, pn, scz)

    lossv = pl.pallas_call(
        _h_body,
        grid=(1,),
        in_specs=[pl.BlockSpec((B, KN * 16), lambda i: (0, 0)),
                  pl.BlockSpec((B, 128), lambda i: (0, 0)),
                  pl.BlockSpec((16, 1, 256), lambda i: (0, 0, 0))],
        out_specs=pl.BlockSpec((1, 1), lambda i: (0, 0)),
        out_shape=jax.ShapeDtypeStruct((1, 1), f32),
    )(dacc.reshape(B, KN * 16), val128, klrows)

    loss = lossv[0, 0]
    p_out = jax.lax.slice(pd, (C, 0), (N, C))
    return p_out, loss
